# baseline (speedup 1.0000x reference)
import functools

import jax
import jax.numpy as jnp
from jax import lax
from jax.experimental import pallas as pl
from jax.experimental.pallas import tpu as pltpu
from jax.experimental.pallas import tpu_sc as plsc

NC = 2
NS = 16
L = 16
NW = NC * NS
CH = 128
NBK = 8


def _rsqrt16(d):
  bits = plsc.bitcast(d, jnp.int32)
  magic = jnp.full((L,), 0x5F3759DF, jnp.int32)
  y = plsc.bitcast(magic - jnp.right_shift(bits, 1), jnp.float32)
  for _ in range(3):
    y = y * (1.5 - 0.5 * d * y * y)
  return y


def _sc_mega(src2, dst2, w2, h, zrow, zvec, npad, nch):
  R = npad // NS
  C = h.shape[1]
  nrows = NW * nch
  deg_blocks = nrows // (NS * NBK)
  prop_blocks = nch // NBK

  deg_rows = nrows // NS
  NV = -(-h.shape[0] // L) * L

  @functools.partial(
      pl.kernel,
      out_type=(
          jax.ShapeDtypeStruct((NC, npad, C), jnp.float32),
          jax.ShapeDtypeStruct((npad,), jnp.float32),
      ),
      mesh=plsc.VectorSubcoreMesh(
          core_axis_name="c", subcore_axis_name="s", num_cores=NC,
          num_subcores=NS),
      compiler_params=pltpu.CompilerParams(needs_layout_passes=False),
      scratch_types=[
          pltpu.VMEM_SHARED((npad, C), jnp.float32),
          pltpu.VMEM_SHARED((npad,), jnp.float32),
          [pltpu.VMEM((NBK, CH), jnp.int32)] * 2,
          [pltpu.VMEM((NBK, CH), jnp.int32)] * 2,
          pltpu.VMEM((NBK, CH), jnp.float32),
          pltpu.VMEM((CH,), jnp.float32),
          pltpu.VMEM((NV,), jnp.float32),
          [pltpu.VMEM((CH, C), jnp.float32)] * 2,
          [pltpu.SemaphoreType.DMA] * 2,
          [pltpu.SemaphoreType.DMA] * 2,
          [pltpu.SemaphoreType.DMA] * 2,
      ],
  )
  def k(src_hbm, dst_hbm, w_hbm, h_hbm, z_hbm, zv_hbm, acc_out, dis_out,
        acc_sh, dis_sh, sidx, didx, w_v, f_v, dis_v, rows, gsem, ssem, msem):
    c = lax.axis_index("c")
    s = lax.axis_index("s")

    pltpu.sync_copy(zv_hbm, dis_sh.at[pl.ds(s * R, R)])
    pltpu.sync_copy(z_hbm, acc_sh.at[pl.ds(s * R, R)])
    plsc.subcore_barrier()

    dbase = s * deg_rows

    def start_deg_meta(kb, q):
      r0 = dbase + kb * NBK
      pltpu.async_copy(dst_hbm.at[pl.ds(r0, NBK)], didx[q], msem[q])
      pltpu.async_copy(w_hbm.at[pl.ds(r0, NBK)], rows[q].at[pl.ds(0, NBK)],
                       msem[q])

    def wait_deg_meta(q):
      pltpu.make_async_copy(
          dst_hbm.at[pl.ds(0, NBK)], didx[q], msem[q]).wait()
      pltpu.make_async_copy(
          w_hbm.at[pl.ds(0, NBK)], rows[q].at[pl.ds(0, NBK)], msem[q]).wait()

    def drain_deg_scatters(q):
      for _ in range(NBK):
        pltpu.make_async_copy(w_hbm.at[0], f_v, ssem[q]).wait()

    start_deg_meta(0, 0)

    @pl.loop(0, deg_blocks, step=2)
    def deg_loop(g0):
      for q in range(2):
        kb = g0 + q
        wait_deg_meta(q)
        for j in range(NBK):
          pltpu.async_copy(rows[q].at[j], dis_sh.at[didx[q].at[j]], ssem[q],
                           add=True)

        @pl.when(kb >= 1)
        def _():
          drain_deg_scatters(1 - q)

        @pl.when(kb + 1 < deg_blocks)
        def _():
          start_deg_meta(kb + 1, 1 - q)

    drain_deg_scatters(1)
    plsc.subcore_barrier()

    pltpu.sync_copy(dis_sh.at[pl.ds(s * R, R)], dis_v.at[pl.ds(0, R)])

    def dis_body(i, carry):
      sl = pl.ds(i * L, L)
      dis_v[sl] = _rsqrt16(dis_v[sl] + 1.0)
      return carry

    lax.fori_loop(0, -(-R // L), dis_body, 0)
    pltpu.sync_copy(dis_v.at[pl.ds(0, R)], dis_sh.at[pl.ds(s * R, R)])

    @pl.when(c == 0)
    def _():
      pltpu.sync_copy(dis_v.at[pl.ds(0, R)], dis_out.at[pl.ds(s * R, R)])

    plsc.subcore_barrier()
    pltpu.sync_copy(dis_sh.at[pl.ds(0, NV)], dis_v)

    base = (c * NS + s) * nch

    def scale_chunk(q, jc, buf):
      for qq in range(CH // L):
        sl = pl.ds(qq * L, L)
        dg = plsc.load_gather(dis_v, [sidx[q][jc, sl]])
        f_v[sl] = w_v[jc, sl] * dg

      @plsc.parallel_loop(0, CH, unroll=4)
      def _(e):
        wb = plsc.load_gather(f_v, [jnp.full((L,), e, jnp.int32)])
        for kk in range(C // L):
          sl = pl.ds(kk * L, L)
          buf[e, sl] = buf[e, sl] * wb

    def start_prop_meta(kb, q):
      r0 = base + kb * NBK
      pltpu.async_copy(src_hbm.at[pl.ds(r0, NBK)], sidx[q], msem[q])
      pltpu.async_copy(dst_hbm.at[pl.ds(r0, NBK)], didx[q], msem[q])

    def wait_prop_meta(q):
      pltpu.make_async_copy(
          src_hbm.at[pl.ds(0, NBK)], sidx[q], msem[q]).wait()
      pltpu.make_async_copy(
          dst_hbm.at[pl.ds(0, NBK)], didx[q], msem[q]).wait()

    start_prop_meta(0, 0)

    @pl.loop(0, prop_blocks, step=2)
    def prop_loop(g0):
      for q in range(2):
        kb = g0 + q
        wait_prop_meta(q)

        @pl.when(kb + 1 < prop_blocks)
        def _():
          start_prop_meta(kb + 1, 1 - q)

        gdesc = [None] * NBK
        sdesc = [None] * NBK
        gdesc[0] = pltpu.async_copy(
            h_hbm.at[sidx[q].at[0]], rows[0], gsem[0])
        pltpu.sync_copy(w_hbm.at[pl.ds(base + kb * NBK, NBK)], w_v)
        for j in range(NBK):
          p = j % 2
          gdesc[j].wait()
          if j + 1 < NBK:
            if j >= 1:
              sdesc[j - 1].wait()
            gdesc[j + 1] = pltpu.async_copy(
                h_hbm.at[sidx[q].at[j + 1]], rows[1 - p], gsem[1 - p])
          scale_chunk(q, j, rows[p])
          sdesc[j] = pltpu.async_copy(
              rows[p], acc_sh.at[didx[q].at[j]], ssem[p], add=True)
        sdesc[NBK - 2].wait()
        sdesc[NBK - 1].wait()

    plsc.subcore_barrier()
    pltpu.sync_copy(acc_sh.at[pl.ds(s * R, R)], acc_out.at[c, pl.ds(s * R, R)])

  return k(src2, dst2, w2, h, zrow, zvec)


def _tc_actnorm(x, acc, discol, W, brow, grow, btrow):
  N, C = x.shape
  inv_n = 1.0 / float(N)

  def body(x_ref, a_ref, d_ref, w_ref, b_ref, gm_ref, bt_ref, y_ref):
    dis = d_ref[...]
    z = a_ref[0] + a_ref[1] + dis * x_ref[...]
    h = jnp.dot(z, w_ref[...], preferred_element_type=jnp.float32,
                precision=lax.Precision.HIGHEST)
    out = dis * h + b_ref[...]
    act = jnp.where(out > 0, out, 0.1 * out)
    ssum = jnp.sum(act, axis=0, keepdims=True)
    ssq = jnp.sum(act * act, axis=0, keepdims=True)
    mean = ssum * inv_n
    var = ssq * inv_n - mean * mean
    inv = lax.rsqrt(var + 1e-5)
    y_ref[...] = (act - mean) * (inv * gm_ref[...]) + bt_ref[...]

  return pl.pallas_call(
      body,
      grid=(1,),
      in_specs=[
          pl.BlockSpec((N, C), lambda i: (0, 0)),
          pl.BlockSpec((NC, N, C), lambda i: (0, 0, 0)),
          pl.BlockSpec((N, 1), lambda i: (0, 0)),
          pl.BlockSpec((C, C), lambda i: (0, 0)),
          pl.BlockSpec((1, C), lambda i: (0, 0)),
          pl.BlockSpec((1, C), lambda i: (0, 0)),
          pl.BlockSpec((1, C), lambda i: (0, 0)),
      ],
      out_specs=pl.BlockSpec((N, C), lambda i: (i, 0)),
      out_shape=jax.ShapeDtypeStruct((N, C), jnp.float32),
  )(x, acc, discol, W, brow, grow, btrow)


def kernel(x, edge_index, edge_attr, W, b, gamma, beta):
  N, C = x.shape
  E = edge_attr.shape[0]

  src = edge_index[0].astype(jnp.int32)
  dst = edge_index[1].astype(jnp.int32)
  w = edge_attr.astype(jnp.float32)

  ept = NW * CH * NBK * 2
  epad = -(-E // ept) * ept
  padn = epad - E
  if padn:
    pidx = jnp.arange(padn, dtype=jnp.int32) % N
    src = jnp.concatenate([src, pidx])
    dst = jnp.concatenate([dst, pidx])
    w = jnp.concatenate([w, jnp.zeros((padn,), jnp.float32)])
  nch = epad // (NW * CH)
  src2 = src.reshape(-1, CH)
  dst2 = dst.reshape(-1, CH)
  w2 = w.reshape(-1, CH)

  R = (-(-N // NS) + 15) // 16 * 16
  npad = R * NS

  zvec = jnp.zeros((R,), jnp.float32)
  zrow = jnp.zeros((R, C), jnp.float32)

  acc, dis = _sc_mega(src2, dst2, w2, x, zrow, zvec, npad, nch)
  y = _tc_actnorm(x, acc, dis.reshape(npad, 1), W, b.reshape(1, C),
                  gamma.reshape(1, C), beta.reshape(1, C))
  return y


# speedup vs baseline: 1.0189x; 1.0189x over previous
import functools

import jax
import jax.numpy as jnp
from jax import lax
from jax.experimental import pallas as pl
from jax.experimental.pallas import tpu as pltpu
from jax.experimental.pallas import tpu_sc as plsc

NC = 2
NS = 16
L = 16
NW = NC * NS
CH = 128
NBK = 8


def _rsqrt16(d):
  bits = plsc.bitcast(d, jnp.int32)
  magic = jnp.full((L,), 0x5F3759DF, jnp.int32)
  y = plsc.bitcast(magic - jnp.right_shift(bits, 1), jnp.float32)
  for _ in range(3):
    y = y * (1.5 - 0.5 * d * y * y)
  return y


def _sc_mega(src2, dst2, w2, h, zrow, zvec, npad, nch):
  R = npad // NS
  C = h.shape[1]
  nrows = NW * nch
  deg_blocks = nrows // (NS * NBK)
  prop_blocks = nch // NBK

  deg_rows = nrows // NS
  NV = -(-h.shape[0] // L) * L

  @functools.partial(
      pl.kernel,
      out_type=(
          jax.ShapeDtypeStruct((NC, npad, C), jnp.float32),
          jax.ShapeDtypeStruct((npad,), jnp.float32),
      ),
      mesh=plsc.VectorSubcoreMesh(
          core_axis_name="c", subcore_axis_name="s", num_cores=NC,
          num_subcores=NS),
      compiler_params=pltpu.CompilerParams(needs_layout_passes=False),
      scratch_types=[
          pltpu.VMEM_SHARED((npad, C), jnp.float32),
          pltpu.VMEM_SHARED((npad,), jnp.float32),
          [pltpu.VMEM((NBK, CH), jnp.int32)] * 2,
          [pltpu.VMEM((NBK, CH), jnp.int32)] * 2,
          pltpu.VMEM((NBK, CH), jnp.float32),
          pltpu.VMEM((CH,), jnp.float32),
          pltpu.VMEM((NV,), jnp.float32),
          [pltpu.VMEM((CH, C), jnp.float32)] * 2,
          [pltpu.SemaphoreType.DMA] * 2,
          [pltpu.SemaphoreType.DMA] * 2,
          [pltpu.SemaphoreType.DMA] * 2,
      ],
  )
  def k(src_hbm, dst_hbm, w_hbm, h_hbm, z_hbm, zv_hbm, acc_out, dis_out,
        acc_sh, dis_sh, sidx, didx, w_v, f_v, dis_v, rows, gsem, ssem, msem):
    c = lax.axis_index("c")
    s = lax.axis_index("s")

    pltpu.sync_copy(zv_hbm, dis_sh.at[pl.ds(s * R, R)])
    pltpu.sync_copy(z_hbm, acc_sh.at[pl.ds(s * R, R)])
    plsc.subcore_barrier()

    dbase = s * deg_rows

    def start_deg_meta(kb, q):
      r0 = dbase + kb * NBK
      pltpu.async_copy(dst_hbm.at[pl.ds(r0, NBK)], didx[q], msem[q])
      pltpu.async_copy(w_hbm.at[pl.ds(r0, NBK)], rows[q].at[pl.ds(0, NBK)],
                       msem[q])

    def wait_deg_meta(q):
      pltpu.make_async_copy(
          dst_hbm.at[pl.ds(0, NBK)], didx[q], msem[q]).wait()
      pltpu.make_async_copy(
          w_hbm.at[pl.ds(0, NBK)], rows[q].at[pl.ds(0, NBK)], msem[q]).wait()

    def drain_deg_scatters(q):
      for _ in range(NBK):
        pltpu.make_async_copy(w_hbm.at[0], f_v, ssem[q]).wait()

    start_deg_meta(0, 0)

    @pl.loop(0, deg_blocks, step=2)
    def deg_loop(g0):
      for q in range(2):
        kb = g0 + q
        wait_deg_meta(q)
        for j in range(NBK):
          pltpu.async_copy(rows[q].at[j], dis_sh.at[didx[q].at[j]], ssem[q],
                           add=True)

        @pl.when(kb >= 1)
        def _():
          drain_deg_scatters(1 - q)

        @pl.when(kb + 1 < deg_blocks)
        def _():
          start_deg_meta(kb + 1, 1 - q)

    drain_deg_scatters(1)
    plsc.subcore_barrier()

    pltpu.sync_copy(dis_sh.at[pl.ds(s * R, R)], dis_v.at[pl.ds(0, R)])

    def dis_body(i, carry):
      sl = pl.ds(i * L, L)
      dis_v[sl] = _rsqrt16(dis_v[sl] + 1.0)
      return carry

    lax.fori_loop(0, -(-R // L), dis_body, 0)
    pltpu.sync_copy(dis_v.at[pl.ds(0, R)], dis_sh.at[pl.ds(s * R, R)])

    @pl.when(c == 0)
    def _():
      pltpu.sync_copy(dis_v.at[pl.ds(0, R)], dis_out.at[pl.ds(s * R, R)])

    plsc.subcore_barrier()
    pltpu.sync_copy(dis_sh.at[pl.ds(0, NV)], dis_v)

    base = (c * NS + s) * nch

    def scale_chunk(q, jc, buf):
      for qq in range(CH // L):
        sl = pl.ds(qq * L, L)
        dg = plsc.load_gather(dis_v, [sidx[q][jc, sl]])
        f_v[sl] = w_v[jc, sl] * dg

      @plsc.parallel_loop(0, CH, unroll=4)
      def _(e):
        wb = plsc.load_gather(f_v, [jnp.full((L,), e, jnp.int32)])
        for kk in range(C // L):
          sl = pl.ds(kk * L, L)
          buf[e, sl] = buf[e, sl] * wb

    def start_prop_meta(kb, q):
      r0 = base + kb * NBK
      pltpu.async_copy(src_hbm.at[pl.ds(r0, NBK)], sidx[q], msem[q])
      pltpu.async_copy(dst_hbm.at[pl.ds(r0, NBK)], didx[q], msem[q])

    def wait_prop_meta(q):
      pltpu.make_async_copy(
          src_hbm.at[pl.ds(0, NBK)], sidx[q], msem[q]).wait()
      pltpu.make_async_copy(
          dst_hbm.at[pl.ds(0, NBK)], didx[q], msem[q]).wait()

    start_prop_meta(0, 0)

    @pl.loop(0, prop_blocks, step=2)
    def prop_loop(g0):
      for q in range(2):
        kb = g0 + q
        wait_prop_meta(q)

        @pl.when(kb + 1 < prop_blocks)
        def _():
          start_prop_meta(kb + 1, 1 - q)

        gdesc = [None] * NBK
        sdesc = [None] * NBK
        gdesc[0] = pltpu.async_copy(
            h_hbm.at[sidx[q].at[0]], rows[0], gsem[0])
        pltpu.sync_copy(w_hbm.at[pl.ds(base + kb * NBK, NBK)], w_v)
        for j in range(NBK):
          p = j % 2
          gdesc[j].wait()
          if j + 1 < NBK:
            if j >= 1:
              sdesc[j - 1].wait()
            gdesc[j + 1] = pltpu.async_copy(
                h_hbm.at[sidx[q].at[j + 1]], rows[1 - p], gsem[1 - p])
          scale_chunk(q, j, rows[p])
          sdesc[j] = pltpu.async_copy(
              rows[p], acc_sh.at[didx[q].at[j]], ssem[p], add=True)
        sdesc[NBK - 2].wait()
        sdesc[NBK - 1].wait()

    plsc.subcore_barrier()
    pltpu.sync_copy(acc_sh.at[pl.ds(s * R, R)], acc_out.at[c, pl.ds(s * R, R)])

  return k(src2, dst2, w2, h, zrow, zvec)


def _tc_actnorm(x, acc, discol, W, brow, grow, btrow):
  N, C = x.shape
  inv_n = 1.0 / float(N)

  def body(x_ref, a_ref, d_ref, w_ref, b_ref, gm_ref, bt_ref, y_ref):
    dis = d_ref[...]
    z = a_ref[0] + a_ref[1] + dis * x_ref[...]
    h = jnp.dot(z, w_ref[...], preferred_element_type=jnp.float32)
    out = dis * h + b_ref[...]
    act = jnp.where(out > 0, out, 0.1 * out)
    ssum = jnp.sum(act, axis=0, keepdims=True)
    ssq = jnp.sum(act * act, axis=0, keepdims=True)
    mean = ssum * inv_n
    var = ssq * inv_n - mean * mean
    inv = lax.rsqrt(var + 1e-5)
    y_ref[...] = (act - mean) * (inv * gm_ref[...]) + bt_ref[...]

  return pl.pallas_call(
      body,
      grid=(1,),
      in_specs=[
          pl.BlockSpec((N, C), lambda i: (0, 0)),
          pl.BlockSpec((NC, N, C), lambda i: (0, 0, 0)),
          pl.BlockSpec((N, 1), lambda i: (0, 0)),
          pl.BlockSpec((C, C), lambda i: (0, 0)),
          pl.BlockSpec((1, C), lambda i: (0, 0)),
          pl.BlockSpec((1, C), lambda i: (0, 0)),
          pl.BlockSpec((1, C), lambda i: (0, 0)),
      ],
      out_specs=pl.BlockSpec((N, C), lambda i: (i, 0)),
      out_shape=jax.ShapeDtypeStruct((N, C), jnp.float32),
  )(x, acc, discol, W, brow, grow, btrow)


def kernel(x, edge_index, edge_attr, W, b, gamma, beta):
  N, C = x.shape
  E = edge_attr.shape[0]

  src = edge_index[0].astype(jnp.int32)
  dst = edge_index[1].astype(jnp.int32)
  w = edge_attr.astype(jnp.float32)

  ept = NW * CH * NBK * 2
  epad = -(-E // ept) * ept
  padn = epad - E
  if padn:
    pidx = jnp.arange(padn, dtype=jnp.int32) % N
    src = jnp.concatenate([src, pidx])
    dst = jnp.concatenate([dst, pidx])
    w = jnp.concatenate([w, jnp.zeros((padn,), jnp.float32)])
  nch = epad // (NW * CH)
  src2 = src.reshape(-1, CH)
  dst2 = dst.reshape(-1, CH)
  w2 = w.reshape(-1, CH)

  R = (-(-N // NS) + 15) // 16 * 16
  npad = R * NS

  zvec = jnp.zeros((R,), jnp.float32)
  zrow = jnp.zeros((R, C), jnp.float32)

  acc, dis = _sc_mega(src2, dst2, w2, x, zrow, zvec, npad, nch)
  y = _tc_actnorm(x, acc, dis.reshape(npad, 1), W, b.reshape(1, C),
                  gamma.reshape(1, C), beta.reshape(1, C))
  return y
